# Initial kernel scaffold; baseline (speedup 1.0000x reference)
import jax
import jax.numpy as jnp
from jax.experimental import pallas as pl


def kernel(x, dutyCycle):
    raise NotImplementedError("write your pallas kernel here")


# speedup vs baseline: 41.0412x; 41.0412x over previous
import jax
import jax.numpy as jnp
from jax.experimental import pallas as pl

_N_UNITS = 4096
_K = 410
_BOOST_STRENGTH = 1.0
_TARGET_DENSITY = float(_K) / _N_UNITS
_ROWS = 256

_INT_MIN = -2147483648


def _kwinners_block(x_ref, dc_ref, o_ref):
    xb = x_ref[...]
    dc = dc_ref[...]
    bf = jnp.exp((_TARGET_DENSITY - dc) * _BOOST_STRENGTH)
    boosted = xb * bf

    t = jax.lax.bitcast_convert_type(boosted, jnp.int32)
    key = t ^ ((t >> 31) & jnp.int32(0x7FFFFFFF))

    imin = jnp.int32(_INT_MIN)
    T = jnp.zeros((xb.shape[0], 1), jnp.int32)
    for b in range(31, -1, -1):
        bit = imin if b == 31 else jnp.int32(1 << b)
        trial = T | bit
        thr = trial ^ imin
        cnt = jnp.sum((key >= thr).astype(jnp.int32), axis=1, keepdims=True)
        T = jnp.where(cnt >= _K, trial, T)

    kth = T ^ imin
    o_ref[...] = jnp.where(key >= kth, xb, 0.0)


def kernel(x, dutyCycle):
    B, N = x.shape
    dc = dutyCycle.reshape(1, N)
    return pl.pallas_call(
        _kwinners_block,
        grid=(B // _ROWS,),
        in_specs=[
            pl.BlockSpec((_ROWS, N), lambda i: (i, 0)),
            pl.BlockSpec((1, N), lambda i: (0, 0)),
        ],
        out_specs=pl.BlockSpec((_ROWS, N), lambda i: (i, 0)),
        out_shape=jax.ShapeDtypeStruct((B, N), x.dtype),
    )(x, dc)
